# Initial kernel scaffold; baseline (speedup 1.0000x reference)
import jax
import jax.numpy as jnp
from jax.experimental import pallas as pl


def kernel(x_tc, times_t, embeddings_tc):
    raise NotImplementedError("write your pallas kernel here")


# speedup vs baseline: 1.3962x; 1.3962x over previous
import functools

import jax
import jax.numpy as jnp
from jax import lax
from jax.experimental import pallas as pl
from jax.experimental.pallas import tpu as pltpu
from jax.experimental.pallas import tpu_sc as plsc

_NUM_CORES = 2
_NUM_SUBCORES = 16
_NW = _NUM_CORES * _NUM_SUBCORES

_SEQ = 4096
_DIM = 1024
_B_PER_W = _SEQ // _NW
_CHUNK = 32
_NCHUNK = _B_PER_W // _CHUNK


def _build_gather():
    mesh = plsc.VectorSubcoreMesh(core_axis_name="c", subcore_axis_name="s")

    @functools.partial(
        pl.kernel,
        mesh=mesh,
        out_type=jax.ShapeDtypeStruct((_SEQ, _DIM), jnp.float32),
        scratch_types=[
            pltpu.VMEM((_B_PER_W,), jnp.int32),
            pltpu.VMEM((_CHUNK, _DIM), jnp.float32),
            pltpu.VMEM((_CHUNK, _DIM), jnp.float32),
            pltpu.SemaphoreType.DMA,
            pltpu.SemaphoreType.DMA,
            pltpu.SemaphoreType.DMA,
            pltpu.SemaphoreType.DMA,
        ],
    )
    def gather_kernel(table_hbm, idx_hbm, out_hbm, idx_v, buf0, buf1,
                      g0, g1, s0, s1):
        wid = lax.axis_index("s") * _NUM_CORES + lax.axis_index("c")
        base = wid * _B_PER_W
        pltpu.sync_copy(idx_hbm.at[pl.ds(base, _B_PER_W)], idx_v)

        bufs = (buf0, buf1)
        gsems = (g0, g1)
        ssems = (s0, s1)
        gathers = [None, None]
        stores = [None, None]
        for c in range(_NCHUNK):
            b = c & 1
            if c >= 2:
                stores[b].wait()
            gathers[b] = pltpu.async_copy(
                table_hbm.at[idx_v.at[pl.ds(c * _CHUNK, _CHUNK)]],
                bufs[b], gsems[b])
            if c >= 1:
                pb = (c - 1) & 1
                gathers[pb].wait()
                stores[pb] = pltpu.async_copy(
                    bufs[pb],
                    out_hbm.at[pl.ds(base + (c - 1) * _CHUNK, _CHUNK)],
                    ssems[pb])
        last = (_NCHUNK - 1) & 1
        gathers[last].wait()
        stores[last] = pltpu.async_copy(
            bufs[last],
            out_hbm.at[pl.ds(base + (_NCHUNK - 1) * _CHUNK, _CHUNK)],
            ssems[last])
        stores[1 - last].wait()
        stores[last].wait()

    return gather_kernel


_gather = _build_gather()


def kernel(x_tc, times_t, embeddings_tc):
    del x_tc
    return _gather(embeddings_tc, times_t.astype(jnp.int32))


# speedup vs baseline: 1.4295x; 1.0238x over previous
import functools

import jax
import jax.numpy as jnp
from jax import lax
from jax.experimental import pallas as pl
from jax.experimental.pallas import tpu as pltpu
from jax.experimental.pallas import tpu_sc as plsc

_NUM_CORES = 2
_NUM_SUBCORES = 16
_NW = _NUM_CORES * _NUM_SUBCORES

_SEQ = 4096
_DIM = 1024
_B_PER_W = _SEQ // _NW
_CHUNK = 32
_NCHUNK = _B_PER_W // _CHUNK
_NBUF = 3


def _build_gather():
    mesh = plsc.VectorSubcoreMesh(core_axis_name="c", subcore_axis_name="s")

    nbuf = min(_NBUF, _NCHUNK)

    @functools.partial(
        pl.kernel,
        mesh=mesh,
        out_type=jax.ShapeDtypeStruct((_SEQ, _DIM), jnp.float32),
        scratch_types=(
            [pltpu.VMEM((_B_PER_W,), jnp.int32)]
            + [pltpu.VMEM((_CHUNK, _DIM), jnp.float32)] * nbuf
            + [pltpu.SemaphoreType.DMA] * (2 * nbuf)
        ),
    )
    def gather_kernel(table_hbm, idx_hbm, out_hbm, idx_v, *scr):
        bufs = scr[:nbuf]
        gsems = scr[nbuf:2 * nbuf]
        ssems = scr[2 * nbuf:]
        wid = lax.axis_index("s") * _NUM_CORES + lax.axis_index("c")
        base = wid * _B_PER_W
        pltpu.sync_copy(idx_hbm.at[pl.ds(base, _B_PER_W)], idx_v)

        gathers = [None] * nbuf
        stores = [None] * nbuf

        def start_gather(c):
            b = c % nbuf
            gathers[b] = pltpu.async_copy(
                table_hbm.at[idx_v.at[pl.ds(c * _CHUNK, _CHUNK)]],
                bufs[b], gsems[b])

        def start_store(c):
            b = c % nbuf
            gathers[b].wait()
            stores[b] = pltpu.async_copy(
                bufs[b], out_hbm.at[pl.ds(base + c * _CHUNK, _CHUNK)],
                ssems[b])

        for c in range(_NCHUNK):
            b = c % nbuf
            if stores[b] is not None:
                stores[b].wait()
            start_gather(c)
            oc = c - (nbuf - 1)
            if oc >= 0:
                start_store(oc)
        for oc in range(max(0, _NCHUNK - nbuf + 1), _NCHUNK):
            start_store(oc)
        for oc in range(max(0, _NCHUNK - nbuf), _NCHUNK):
            stores[oc % nbuf].wait()

    return gather_kernel


_gather = _build_gather()


def kernel(x_tc, times_t, embeddings_tc):
    del x_tc
    return _gather(embeddings_tc, times_t.astype(jnp.int32))


# speedup vs baseline: 1.4500x; 1.0144x over previous
import functools

import jax
import jax.numpy as jnp
from jax import lax
from jax.experimental import pallas as pl
from jax.experimental.pallas import tpu as pltpu
from jax.experimental.pallas import tpu_sc as plsc

_NUM_CORES = 2
_NUM_SUBCORES = 16
_NW = _NUM_CORES * _NUM_SUBCORES

_SEQ = 4096
_DIM = 1024
_B_PER_W = _SEQ // _NW
_CHUNK = 16
_NCHUNK = _B_PER_W // _CHUNK
_NBUF = 6


def _build_gather():
    mesh = plsc.VectorSubcoreMesh(core_axis_name="c", subcore_axis_name="s")

    nbuf = min(_NBUF, _NCHUNK)

    @functools.partial(
        pl.kernel,
        mesh=mesh,
        out_type=jax.ShapeDtypeStruct((_SEQ, _DIM), jnp.float32),
        scratch_types=(
            [pltpu.VMEM((_B_PER_W,), jnp.int32)]
            + [pltpu.VMEM((_CHUNK, _DIM), jnp.float32)] * nbuf
            + [pltpu.SemaphoreType.DMA] * (2 * nbuf)
        ),
    )
    def gather_kernel(table_hbm, idx_hbm, out_hbm, idx_v, *scr):
        bufs = scr[:nbuf]
        gsems = scr[nbuf:2 * nbuf]
        ssems = scr[2 * nbuf:]
        wid = lax.axis_index("s") * _NUM_CORES + lax.axis_index("c")
        base = wid * _B_PER_W
        pltpu.sync_copy(idx_hbm.at[pl.ds(base, _B_PER_W)], idx_v)

        gathers = [None] * nbuf
        stores = [None] * nbuf

        def start_gather(c):
            b = c % nbuf
            gathers[b] = pltpu.async_copy(
                table_hbm.at[idx_v.at[pl.ds(c * _CHUNK, _CHUNK)]],
                bufs[b], gsems[b])

        def start_store(c):
            b = c % nbuf
            gathers[b].wait()
            stores[b] = pltpu.async_copy(
                bufs[b], out_hbm.at[pl.ds(base + c * _CHUNK, _CHUNK)],
                ssems[b])

        for c in range(_NCHUNK):
            b = c % nbuf
            if stores[b] is not None:
                stores[b].wait()
            start_gather(c)
            oc = c - (nbuf - 1)
            if oc >= 0:
                start_store(oc)
        for oc in range(max(0, _NCHUNK - nbuf + 1), _NCHUNK):
            start_store(oc)
        for oc in range(max(0, _NCHUNK - nbuf), _NCHUNK):
            stores[oc % nbuf].wait()

    return gather_kernel


_gather = _build_gather()


def kernel(x_tc, times_t, embeddings_tc):
    del x_tc
    return _gather(embeddings_tc, times_t.astype(jnp.int32))
